# Initial kernel scaffold; baseline (speedup 1.0000x reference)
import jax
import jax.numpy as jnp
from jax.experimental import pallas as pl


def kernel(x, ids, table, W1, b1, W2, b2):
    raise NotImplementedError("write your pallas kernel here")


# speedup vs baseline: 9.2854x; 9.2854x over previous
import functools

import jax
import jax.numpy as jnp
from jax import lax
from jax.experimental import pallas as pl
from jax.experimental.pallas import tpu as pltpu
from jax.experimental.pallas import tpu_sc as plsc

B, L, V, D, H = 4096, 50, 100000, 128, 512

NC, NS = 2, 16
NW = NC * NS
SPT = B // NW
ROWS = 2 * L
CHUNKS = SPT // 2
SC_SAMPLES = NS * SPT


def _pooled_sum_sc(ids2, table, dst_idx, zeros):
    mesh = plsc.VectorSubcoreMesh(core_axis_name="c", subcore_axis_name="s")

    @functools.partial(
        pl.kernel,
        mesh=mesh,
        out_type=jax.ShapeDtypeStruct((B, D), jnp.float32),
        scratch_types=[
            pltpu.VMEM((CHUNKS, ROWS), jnp.int32),
            pltpu.VMEM((CHUNKS, ROWS), jnp.int32),
            pltpu.VMEM((ROWS, D), jnp.float32),
            pltpu.VMEM((ROWS, D), jnp.float32),
            pltpu.VMEM_SHARED((SC_SAMPLES, D), jnp.float32),
            pltpu.SemaphoreType.DMA,
            pltpu.SemaphoreType.DMA,
        ],
    )
    def k(ids_hbm, table_hbm, dst_hbm, zeros_hbm, out_hbm,
          ids_v, dst_v, buf_a, buf_b, acc, sem_a, sem_b):
        c = lax.axis_index("c")
        s = lax.axis_index("s")
        row_base = c * (NS * CHUNKS) + s * CHUNKS
        abase = s * SPT

        pltpu.sync_copy(ids_hbm.at[pl.ds(row_base, CHUNKS)], ids_v)
        pltpu.sync_copy(dst_hbm.at[s], dst_v)
        pltpu.sync_copy(zeros_hbm.at[pl.ds(abase, SPT)],
                        acc.at[pl.ds(abase, SPT)])

        pltpu.async_copy(table_hbm.at[ids_v.at[0]], buf_a, sem_a)
        pltpu.async_copy(table_hbm.at[ids_v.at[1]], buf_b, sem_b)

        def body(p, carry):
            j0 = 2 * p
            pltpu.make_async_copy(table_hbm.at[ids_v.at[j0]], buf_a, sem_a).wait()
            pltpu.sync_copy(buf_a, acc.at[dst_v.at[j0]], add=True)
            pltpu.async_copy(table_hbm.at[ids_v.at[j0 + 2]], buf_a, sem_a)
            pltpu.make_async_copy(table_hbm.at[ids_v.at[j0 + 1]], buf_b, sem_b).wait()
            pltpu.sync_copy(buf_b, acc.at[dst_v.at[j0 + 1]], add=True)
            pltpu.async_copy(table_hbm.at[ids_v.at[j0 + 3]], buf_b, sem_b)
            return carry

        lax.fori_loop(0, CHUNKS // 2 - 1, body, 0)

        pltpu.make_async_copy(table_hbm.at[ids_v.at[CHUNKS - 2]], buf_a, sem_a).wait()
        pltpu.sync_copy(buf_a, acc.at[dst_v.at[CHUNKS - 2]], add=True)
        pltpu.make_async_copy(table_hbm.at[ids_v.at[CHUNKS - 1]], buf_b, sem_b).wait()
        pltpu.sync_copy(buf_b, acc.at[dst_v.at[CHUNKS - 1]], add=True)

        gbase = c * SC_SAMPLES + abase
        pltpu.sync_copy(acc.at[pl.ds(abase, SPT)],
                        out_hbm.at[pl.ds(gbase, SPT)])

    return k(ids2, table, dst_idx, zeros)


BM = 512


def _mlp_tc(x, pooled_sum, w1a, w1b, b1, w2, b2):
    def body(x_ref, p_ref, w1a_ref, w1b_ref, b1_ref, w2_ref, b2_ref, o_ref):
        xb = x_ref[...]
        pb = p_ref[...] * (1.0 / L)
        z = jnp.dot(xb, w1a_ref[...], preferred_element_type=jnp.float32)
        z = z + jnp.dot(pb, w1b_ref[...], preferred_element_type=jnp.float32)
        z = jnp.maximum(z + b1_ref[...], 0.0)
        o = jnp.dot(z, w2_ref[...], preferred_element_type=jnp.float32)
        o_ref[...] = o + b2_ref[...]

    return pl.pallas_call(
        body,
        grid=(B // BM,),
        in_specs=[
            pl.BlockSpec((BM, D), lambda i: (i, 0)),
            pl.BlockSpec((BM, D), lambda i: (i, 0)),
            pl.BlockSpec((D, H), lambda i: (0, 0)),
            pl.BlockSpec((D, H), lambda i: (0, 0)),
            pl.BlockSpec((1, H), lambda i: (0, 0)),
            pl.BlockSpec((H, D), lambda i: (0, 0)),
            pl.BlockSpec((1, D), lambda i: (0, 0)),
        ],
        out_specs=pl.BlockSpec((BM, D), lambda i: (i, 0)),
        out_shape=jax.ShapeDtypeStruct((B, D), jnp.float32),
    )(x, pooled_sum, w1a, w1b, b1.reshape(1, H), w2, b2.reshape(1, D))


def kernel(x, ids, table, W1, b1, W2, b2):
    ids2 = ids.astype(jnp.int32).reshape(B // 2, ROWS)
    dst_idx = (
        jnp.arange(NS, dtype=jnp.int32)[:, None] * SPT
        + jnp.repeat(jnp.arange(SPT, dtype=jnp.int32), L)[None, :]
    ).reshape(NS, CHUNKS, ROWS)
    zeros = jnp.zeros((SC_SAMPLES, D), jnp.float32)
    pooled_sum = _pooled_sum_sc(ids2, table, dst_idx, zeros)
    return _mlp_tc(x, pooled_sum, W1[:D], W1[D:], b1, W2, b2)
